# Initial kernel scaffold; baseline (speedup 1.0000x reference)
import functools

import jax
import jax.numpy as jnp
from jax import lax
from jax.experimental import pallas as pl
from jax.experimental.pallas import tpu as pltpu
from jax.experimental.pallas import tpu_sc as plsc

BATCH = 16384
N_FIELDS = 26
FIELD_SIZE = 100000
LANES = 16


def _build_kernel(num_cores, num_workers, rows_per_worker):
    n_chunks = rows_per_worker // LANES
    n_quarters = rows_per_worker // 128
    n_dmas = N_FIELDS * n_quarters
    mesh = plsc.VectorSubcoreMesh(core_axis_name="c", subcore_axis_name="s")

    @functools.partial(
        pl.kernel,
        out_type=jax.ShapeDtypeStruct((BATCH,), jnp.float32),
        mesh=mesh,
        scratch_types=[
            pltpu.VMEM((rows_per_worker, N_FIELDS), jnp.int32),
            pltpu.VMEM((n_dmas, 128), jnp.int32),
            pltpu.VMEM((n_dmas, 128), jnp.float32),
            pltpu.VMEM((rows_per_worker,), jnp.float32),
            pltpu.VMEM((LANES,), jnp.float32),
            pltpu.SemaphoreType.DMA,
        ],
    )
    def ker(x_hbm, w_hbm, bias_hbm, out_hbm, x_v, idx_v, emb_v, out_v,
            bias_v, sem):
        wid = lax.axis_index("s") * num_cores + lax.axis_index("c")
        base = wid * rows_per_worker

        pltpu.sync_copy(x_hbm.at[pl.ds(base, rows_per_worker)], x_v)
        pltpu.sync_copy(bias_hbm, bias_v)

        lane = lax.iota(jnp.int32, (LANES,))

        def build(c, carry):
            rows = lane + c * LANES
            q = c // 8
            s16 = (c % 8) * LANES
            for f in range(N_FIELDS):
                col = jnp.full((LANES,), f, jnp.int32)
                vals = plsc.load_gather(x_v, [rows, col])
                idx_v[f * n_quarters + q, pl.ds(s16, LANES)] = (
                    vals + f * FIELD_SIZE)
            return carry

        lax.fori_loop(0, n_chunks, build, 0)

        wave = 8
        n_waves = n_dmas // wave

        def fire(w):
            for b in range(wave):
                j = w * wave + b
                pltpu.async_copy(w_hbm.at[idx_v.at[j]], emb_v.at[j], sem)

        def drain(w):
            for b in range(wave):
                j = w * wave + b
                pltpu.make_async_copy(
                    w_hbm.at[idx_v.at[j]], emb_v.at[j], sem).wait()

        def dma_step(w, carry):
            @pl.when(w + 1 < n_waves)
            def _fire():
                fire(w + 1)

            drain(w)
            return carry

        fire(0)
        lax.fori_loop(0, n_waves, dma_step, 0)

        bias_vec = bias_v[...]

        def reduce(c, carry):
            q = c // 8
            s16 = (c % 8) * LANES
            acc = bias_vec
            for f in range(N_FIELDS):
                acc = acc + emb_v[f * n_quarters + q, pl.ds(s16, LANES)]
            out_v[pl.ds(c * LANES, LANES)] = acc
            return carry

        lax.fori_loop(0, n_chunks, reduce, 0)

        pltpu.sync_copy(out_v, out_hbm.at[pl.ds(base, rows_per_worker)])

    return ker


def kernel(x, weights, bias):
    info = plsc.get_sparse_core_info()
    num_workers = info.num_cores * info.num_subcores
    rows_per_worker = BATCH // num_workers
    ker = _build_kernel(info.num_cores, num_workers, rows_per_worker)
    w_flat = weights.reshape(-1)
    bias16 = jnp.broadcast_to(bias, (LANES,))
    out = ker(x.astype(jnp.int32), w_flat, bias16)
    return out.reshape(BATCH, 1)


# speedup vs baseline: 1.1425x; 1.1425x over previous
import functools

import jax
import jax.numpy as jnp
from jax import lax
from jax.experimental import pallas as pl
from jax.experimental.pallas import tpu as pltpu
from jax.experimental.pallas import tpu_sc as plsc

BATCH = 16384
N_FIELDS = 26
FIELD_SIZE = 100000
LANES = 16
CHUNK = 128


def _build_kernel(num_cores, num_workers, rows_per_worker):
    n_groups = rows_per_worker // CHUNK
    n_dmas = N_FIELDS * n_groups
    vpd = CHUNK // LANES
    mesh = plsc.VectorSubcoreMesh(core_axis_name="c", subcore_axis_name="s")

    @functools.partial(
        pl.kernel,
        out_type=jax.ShapeDtypeStruct((BATCH,), jnp.float32),
        mesh=mesh,
        compiler_params=pltpu.CompilerParams(needs_layout_passes=False),
        scratch_types=[
            pltpu.VMEM((rows_per_worker * N_FIELDS,), jnp.int32),
            pltpu.VMEM((n_dmas, CHUNK), jnp.int32),
            pltpu.VMEM((n_dmas, CHUNK), jnp.float32),
            pltpu.VMEM((rows_per_worker,), jnp.float32),
            pltpu.VMEM((LANES,), jnp.float32),
            pltpu.SemaphoreType.DMA,
        ],
    )
    def ker(x_hbm, w_hbm, bias_hbm, out_hbm, x_v, idx_v, emb_v, out_v,
            bias_v, sem):
        wid = lax.axis_index("s") * num_cores + lax.axis_index("c")
        base = wid * rows_per_worker

        pltpu.sync_copy(
            x_hbm.at[pl.ds(base * N_FIELDS, rows_per_worker * N_FIELDS)], x_v)
        pltpu.sync_copy(bias_hbm, bias_v)

        lane = lax.iota(jnp.int32, LANES)

        def build_c(c, carry):
            flat0 = (lane + c * LANES) * N_FIELDS
            g = c // vpd
            s16 = (c % vpd) * LANES
            for f in range(N_FIELDS):
                vals = plsc.load_gather(x_v, [flat0 + f])
                idx_v[g * N_FIELDS + f, pl.ds(s16, LANES)] = (
                    vals + f * FIELD_SIZE)
            return carry

        def build(g):
            lax.fori_loop(g * vpd, (g + 1) * vpd, build_c, 0)

        def fire(g):
            for f in range(N_FIELDS):
                j = g * N_FIELDS + f
                pltpu.async_copy(w_hbm.at[idx_v.at[j]], emb_v.at[j], sem)

        def drain(g):
            for f in range(N_FIELDS):
                j = g * N_FIELDS + f
                pltpu.make_async_copy(
                    w_hbm.at[idx_v.at[j]], emb_v.at[j], sem).wait()

        bias_vec = bias_v[...]

        def reduce_c(c, carry):
            g = c // vpd
            s16 = (c % vpd) * LANES
            acc = bias_vec
            for f in range(N_FIELDS):
                acc = acc + emb_v[g * N_FIELDS + f, pl.ds(s16, LANES)]
            out_v[pl.ds(c * LANES, LANES)] = acc
            return carry

        def reduce(g):
            lax.fori_loop(g * vpd, (g + 1) * vpd, reduce_c, 0)

        build(0)
        fire(0)

        def step(g, carry):
            @pl.when(g + 1 < n_groups)
            def _ahead():
                build(g + 1)
                fire(g + 1)

            drain(g)
            reduce(g)
            return carry

        lax.fori_loop(0, n_groups, step, 0)

        pltpu.sync_copy(out_v, out_hbm.at[pl.ds(base, rows_per_worker)])

    return ker


def kernel(x, weights, bias):
    info = plsc.get_sparse_core_info()
    num_workers = info.num_cores * info.num_subcores
    rows_per_worker = BATCH // num_workers
    ker = _build_kernel(info.num_cores, num_workers, rows_per_worker)
    w_flat = weights.reshape(-1)
    bias16 = jnp.broadcast_to(bias, (LANES,))
    out = ker(x.astype(jnp.int32).reshape(-1), w_flat, bias16)
    return out.reshape(BATCH, 1)
